# baseline (speedup 1.0000x reference)
import functools

import numpy as np

import jax
import jax.numpy as jnp
from jax import lax
from jax.experimental import pallas as pl
from jax.experimental.pallas import tpu as pltpu
from jax.experimental.pallas import tpu_sc as plsc

S = 2048
D = 1024
E = 8
K = 2
P = S * K
NC, NS, L = 2, 16, 16
NW = NC * NS
TW = S // NW
T = 512
NT = P // T
G = NT + E - 1

NEG = -3.0e38


def _lane():
    return lax.broadcasted_iota(jnp.int32, (L,), 0)


def _prefix_total(x, buf):
    acc = x
    for k in (1, 2, 4, 8):
        buf[pl.ds(L, L)] = acc
        acc = acc + buf[pl.ds(L - k, L)]
    suf = x
    for k in (1, 2, 4, 8):
        buf[pl.ds(L, L)] = suf
        suf = suf + buf[pl.ds(L + k, L)]
    return acc, acc + suf - x


def _sc_routing_body(rs_hbm, e0_hbm, e1_hbm, w0_hbm, w1_hbm, cnt_hbm,
                     rs_v, e0_v, e1_v, w0_v, w1_v, cnt_v, sbuf):
    wid = lax.axis_index("s") * NC + lax.axis_index("c")
    base = wid * TW
    pltpu.sync_copy(rs_hbm.at[wid], rs_v)
    lane = _lane()
    zero = jnp.zeros((L,), jnp.int32)
    sbuf[pl.ds(0, L)] = zero
    sbuf[pl.ds(2 * L, L)] = zero
    cnt_acc = zero
    for c in range(TW // L):
        sl = pl.ds(c * L, L)
        v0 = rs_v[0, sl]
        i0 = jnp.zeros((L,), jnp.int32)
        for e in range(1, E):
            cand = rs_v[e, sl]
            m = cand > v0
            v0 = jnp.where(m, cand, v0)
            i0 = jnp.where(m, e, i0)
        v1 = jnp.full((L,), NEG, jnp.float32)
        i1 = jnp.zeros((L,), jnp.int32)
        for e in range(E):
            cand = rs_v[e, sl]
            m = (cand > v1) & (i0 != e)
            v1 = jnp.where(m, cand, v1)
            i1 = jnp.where(m, e, i1)
        sc = 1.0 / (v0 + v1 + 1e-08)
        e0_v[sl] = i0
        e1_v[sl] = i1
        w0_v[sl] = v0 * sc
        w1_v[sl] = v1 * sc
        for e in range(E):
            mi = jnp.where(i0 == e, 1, 0) + jnp.where(i1 == e, 1, 0)
            _, tot = _prefix_total(mi, sbuf)
            cnt_acc += jnp.where(lane == e, tot, 0)
    cnt_v[...] = cnt_acc
    pltpu.sync_copy(e0_v, e0_hbm.at[pl.ds(base, TW)])
    pltpu.sync_copy(e1_v, e1_hbm.at[pl.ds(base, TW)])
    pltpu.sync_copy(w0_v, w0_hbm.at[pl.ds(base, TW)])
    pltpu.sync_copy(w1_v, w1_hbm.at[pl.ds(base, TW)])
    pltpu.sync_copy(cnt_v, cnt_hbm.at[wid])


def _sc_dispatch_body(x_hbm, e0_hbm, e1_hbm, w0_hbm, w1_hbm, cnt_hbm, widtab_hbm,
                      xs_hbm, wp_hbm, p0_hbm, p1_hbm, cnt8_hbm,
                      cnt_m, e0_v, e1_v, p0_v, p1_v, widrow_v, sbuf,
                      w0pad, w1pad, wrow0, wrow1, xrow_v, cnt8_v, sem0, sem1):
    wid = lax.axis_index("s") * NC + lax.axis_index("c")
    base = wid * TW
    gat = pltpu.async_copy(x_hbm.at[pl.ds(base, TW)], xrow_v, sem0)
    pltpu.sync_copy(cnt_hbm, cnt_m)
    pltpu.sync_copy(e0_hbm.at[pl.ds(base, TW)], e0_v)
    pltpu.sync_copy(e1_hbm.at[pl.ds(base, TW)], e1_v)
    pltpu.sync_copy(w0_hbm.at[pl.ds(base, TW)], w0pad.at[pl.ds(0, TW)])
    pltpu.sync_copy(w1_hbm.at[pl.ds(base, TW)], w1pad.at[pl.ds(0, TW)])
    pltpu.sync_copy(widtab_hbm.at[wid], widrow_v)

    lane = _lane()
    zero = jnp.zeros((L,), jnp.int32)
    sbuf[pl.ds(0, L)] = zero
    sbuf[pl.ds(2 * L, L)] = zero
    w0pad[pl.ds(TW, L)] = jnp.zeros((L,), jnp.float32)
    w1pad[pl.ds(TW, L)] = jnp.zeros((L,), jnp.float32)

    widrow = widrow_v[...]
    total = zero
    prior = zero
    for w in range(NW):
        row = cnt_m[w, :]
        total += row
        prior += jnp.where(widrow > w, row, 0)
    cnt8_v[...] = total
    @pl.when(wid == 0)
    def _write_counts():
        pltpu.sync_copy(cnt8_v, cnt8_hbm)
    pref, _ = _prefix_total(total, sbuf)
    start = (pref - total) + prior
    start_sp = []
    for e in range(E):
        _, sp = _prefix_total(jnp.where(lane == e, start, 0), sbuf)
        start_sp.append(sp)

    carry = [zero for _ in range(E)]
    for c in range(TW // L):
        sl = pl.ds(c * L, L)
        e0c = e0_v[sl]
        e1c = e1_v[sl]
        rank0 = zero
        rank1 = zero
        s0 = zero
        s1 = zero
        for e in range(E):
            m0 = e0c == e
            m1 = e1c == e
            mi = jnp.where(m0, 1, 0) + jnp.where(m1, 1, 0)
            pref_e, tot_e = _prefix_total(mi, sbuf)
            ex = pref_e - mi + carry[e]
            rank0 += jnp.where(m0, ex, 0)
            rank1 += jnp.where(m1, ex, 0)
            s0 += jnp.where(m0, start_sp[e], 0)
            s1 += jnp.where(m1, start_sp[e], 0)
            carry[e] = carry[e] + tot_e
        p0_v[sl] = s0 + rank0
        p1_v[sl] = s1 + rank1

    for i in range(TW):
        wrow0[i, pl.ds(0, L)] = w0pad[pl.ds(i, L)]
        wrow1[i, pl.ds(0, L)] = w1pad[pl.ds(i, L)]

    pltpu.sync_copy(p0_v, p0_hbm.at[pl.ds(base, TW)])
    pltpu.sync_copy(p1_v, p1_hbm.at[pl.ds(base, TW)])
    gat.wait()
    sc0 = pltpu.async_copy(xrow_v, xs_hbm.at[p0_v], sem0)
    sc1 = pltpu.async_copy(xrow_v, xs_hbm.at[p1_v], sem1)
    sc0.wait()
    sc1.wait()
    sw0 = pltpu.async_copy(wrow0, wp_hbm.at[p0_v], sem0)
    sw1 = pltpu.async_copy(wrow1, wp_hbm.at[p1_v], sem1)
    sw0.wait()
    sw1.wait()


def _tc_gmm_body(meta_ref, xs_ref, w_ref, b_ref, wp_ref, ys_ref, wbf_ref):
    g = pl.program_id(0)
    eg = meta_ref[0, g]
    lo = meta_ref[2, g]
    hi = meta_ref[3, g]

    @pl.when(g == 0)
    def _cast_w():
        wbf_ref[...] = w_ref[...].astype(jnp.bfloat16)

    x32 = xs_ref[...]
    xa = pltpu.bitcast(x32 << 16, jnp.float32)
    xb = pltpu.bitcast(x32 & jnp.int32(-65536), jnp.float32)
    y = jnp.dot(xa.astype(jnp.bfloat16), wbf_ref[eg, : D // 2, :],
                preferred_element_type=jnp.float32)
    y = y + jnp.dot(xb.astype(jnp.bfloat16), wbf_ref[eg, D // 2 :, :],
                    preferred_element_type=jnp.float32)
    y = (y + b_ref[eg]) * wp_ref[:, 0:1]
    ri = lax.broadcasted_iota(jnp.int32, (T, 1), 0)
    keep = (ri >= lo) & (ri < hi)
    ys_ref[...] = jnp.where(keep, y, ys_ref[...])


_CH = 16
_NCH = TW // _CH


def _sc_combine_body(ys_hbm, p0_hbm, p1_hbm, out_hbm,
                     p0a, p0b, p1a, p1b, r0a, r0b, r1a, r1b, oa, ob,
                     sga, sgb, soa, sob):
    wid = lax.axis_index("s") * NC + lax.axis_index("c")
    base = wid * TW
    p0s = (p0a, p0b)
    p1s = (p1a, p1b)
    r0s = (r0a, r0b)
    r1s = (r1a, r1b)
    outs = (oa, ob)
    gsems = (sga, sgb)
    osems = (soa, sob)

    def issue(c):
        k = c % 2
        cbase = base + c * _CH
        pltpu.sync_copy(p0_hbm.at[pl.ds(cbase, _CH)], p0s[k])
        pltpu.sync_copy(p1_hbm.at[pl.ds(cbase, _CH)], p1s[k])
        g0 = pltpu.async_copy(ys_hbm.at[p0s[k]], r0s[k], gsems[k])
        g1 = pltpu.async_copy(ys_hbm.at[p1s[k]], r1s[k], gsems[k])
        return g0, g1

    pend = issue(0)
    owaits = [None, None]
    for c in range(_NCH):
        k = c % 2
        cur = pend
        if c + 1 < _NCH:
            pend = issue(c + 1)
        cur[0].wait()
        cur[1].wait()
        if owaits[k] is not None:
            owaits[k].wait()

        def tok(t, carry):
            for j in range(D // L):
                sl = pl.ds(j * L, L)
                outs[k][t, sl] = r0s[k][t, sl] + r1s[k][t, sl]
            return carry

        lax.fori_loop(0, _CH, tok, 0)
        owaits[k] = pltpu.async_copy(
            outs[k], out_hbm.at[pl.ds(base + c * _CH, _CH)], osems[k])
    for w in owaits:
        if w is not None:
            w.wait()


def _gmm_metadata(counts):
    offsets = jnp.concatenate(
        [jnp.zeros((1,), jnp.int32), jnp.cumsum(counts, dtype=jnp.int32)])
    t_start = offsets[:E] // T
    t_end = jnp.where(counts > 0, (offsets[1:] - 1) // T, t_start - 1)
    nsteps = jnp.maximum(t_end - t_start + 1, 0)
    cum = jnp.concatenate(
        [jnp.zeros((1,), jnp.int32), jnp.cumsum(nsteps, dtype=jnp.int32)])
    g = jnp.arange(G, dtype=jnp.int32)
    e_of_g = jnp.sum((g[:, None] >= cum[None, 1:]).astype(jnp.int32), axis=1)
    valid = g < cum[E]
    e_g = jnp.clip(e_of_g, 0, E - 1)
    t_g = t_start[e_g] + (g - cum[e_g])
    t_g = jnp.where(valid, t_g, NT - 1)
    lo = jnp.clip(offsets[e_g] - t_g * T, 0, T)
    hi = jnp.clip(offsets[e_g + 1] - t_g * T, 0, T)
    lo = jnp.where(valid, lo, 0)
    hi = jnp.where(valid, hi, 0)
    return jnp.stack([e_g, t_g, lo, hi]).astype(jnp.int32)


_sc_mesh = plsc.VectorSubcoreMesh(
    core_axis_name="c", subcore_axis_name="s", num_cores=NC, num_subcores=NS)

_routing = pl.kernel(
    _sc_routing_body,
    out_type=[
        jax.ShapeDtypeStruct((S,), jnp.int32),
        jax.ShapeDtypeStruct((S,), jnp.int32),
        jax.ShapeDtypeStruct((S,), jnp.float32),
        jax.ShapeDtypeStruct((S,), jnp.float32),
        jax.ShapeDtypeStruct((NW, L), jnp.int32),
    ],
    mesh=_sc_mesh,
    scratch_types=[
        pltpu.VMEM((E, TW), jnp.float32),
        pltpu.VMEM((TW,), jnp.int32),
        pltpu.VMEM((TW,), jnp.int32),
        pltpu.VMEM((TW,), jnp.float32),
        pltpu.VMEM((TW,), jnp.float32),
        pltpu.VMEM((L,), jnp.int32),
        pltpu.VMEM((3 * L,), jnp.int32),
    ],
)

_dispatch = pl.kernel(
    _sc_dispatch_body,
    out_type=[
        jax.ShapeDtypeStruct((P, D // 2), jnp.int32),
        jax.ShapeDtypeStruct((P, 128), jnp.float32),
        jax.ShapeDtypeStruct((S,), jnp.int32),
        jax.ShapeDtypeStruct((S,), jnp.int32),
        jax.ShapeDtypeStruct((L,), jnp.int32),
    ],
    mesh=_sc_mesh,
    scratch_types=[
        pltpu.VMEM((NW, L), jnp.int32),
        pltpu.VMEM((TW,), jnp.int32),
        pltpu.VMEM((TW,), jnp.int32),
        pltpu.VMEM((TW,), jnp.int32),
        pltpu.VMEM((TW,), jnp.int32),
        pltpu.VMEM((L,), jnp.int32),
        pltpu.VMEM((3 * L,), jnp.int32),
        pltpu.VMEM((TW + L,), jnp.float32),
        pltpu.VMEM((TW + L,), jnp.float32),
        pltpu.VMEM((TW, 128), jnp.float32),
        pltpu.VMEM((TW, 128), jnp.float32),
        pltpu.VMEM((TW, D // 2), jnp.int32),
        pltpu.VMEM((L,), jnp.int32),
        pltpu.SemaphoreType.DMA,
        pltpu.SemaphoreType.DMA,
    ],
)

_combine = pl.kernel(
    _sc_combine_body,
    out_type=[jax.ShapeDtypeStruct((S, D), jnp.float32)],
    mesh=_sc_mesh,
    scratch_types=[
        pltpu.VMEM((_CH,), jnp.int32),
        pltpu.VMEM((_CH,), jnp.int32),
        pltpu.VMEM((_CH,), jnp.int32),
        pltpu.VMEM((_CH,), jnp.int32),
        pltpu.VMEM((_CH, D), jnp.float32),
        pltpu.VMEM((_CH, D), jnp.float32),
        pltpu.VMEM((_CH, D), jnp.float32),
        pltpu.VMEM((_CH, D), jnp.float32),
        pltpu.VMEM((_CH, D), jnp.float32),
        pltpu.VMEM((_CH, D), jnp.float32),
        pltpu.SemaphoreType.DMA,
        pltpu.SemaphoreType.DMA,
        pltpu.SemaphoreType.DMA,
        pltpu.SemaphoreType.DMA,
    ],
)

_WIDTAB = np.tile(np.arange(NW, dtype=np.int32)[:, None], (1, L))


@jax.jit
def kernel(x, routing_scores, expert_w, expert_b):
    B = x.shape[0]
    x2 = x.reshape(S, D)
    rs_w = routing_scores.reshape(NW, TW, E).transpose(0, 2, 1)

    e0, e1, w0, w1, cnt_all = _routing(rs_w)

    x_bf = x2.astype(jnp.bfloat16)
    x_bfp = lax.bitcast_convert_type(
        jnp.stack([x_bf[:, :D // 2], x_bf[:, D // 2:]], axis=-1), jnp.int32)
    xs, wp, pos0, pos1, cnt8 = _dispatch(x_bfp, e0, e1, w0, w1, cnt_all, _WIDTAB)
    counts = cnt8[:E]
    meta = _gmm_metadata(counts)

    ys = pl.pallas_call(
        _tc_gmm_body,
        grid_spec=pltpu.PrefetchScalarGridSpec(
            num_scalar_prefetch=1,
            grid=(G,),
            in_specs=[
                pl.BlockSpec((T, D // 2), lambda g, m: (m[1, g], 0)),
                pl.BlockSpec((E, D, D), lambda g, m: (0, 0, 0)),
                pl.BlockSpec((E, 1, D), lambda g, m: (0, 0, 0)),
                pl.BlockSpec((T, 128), lambda g, m: (m[1, g], 0)),
            ],
            out_specs=pl.BlockSpec((T, D), lambda g, m: (m[1, g], 0)),
            scratch_shapes=[pltpu.VMEM((E, D, D), jnp.bfloat16)],
        ),
        out_shape=jax.ShapeDtypeStruct((P, D), jnp.float32),
        compiler_params=pltpu.CompilerParams(
            dimension_semantics=("arbitrary",),
        ),
    )(meta, xs, expert_w, expert_b.reshape(E, 1, D), wp)

    (out,) = _combine(ys, pos0, pos1)
    return out.reshape(B, S, D), counts


# speedup vs baseline: 1.0104x; 1.0104x over previous
import functools

import numpy as np

import jax
import jax.numpy as jnp
from jax import lax
from jax.experimental import pallas as pl
from jax.experimental.pallas import tpu as pltpu
from jax.experimental.pallas import tpu_sc as plsc

S = 2048
D = 1024
E = 8
K = 2
P = S * K
NC, NS, L = 2, 16, 16
NW = NC * NS
TW = S // NW
T = 512
NT = P // T
G = NT + E - 1

NEG = -3.0e38


def _lane():
    return lax.broadcasted_iota(jnp.int32, (L,), 0)


def _prefix_total(x, buf):
    acc = x
    for k in (1, 2, 4, 8):
        buf[pl.ds(L, L)] = acc
        acc = acc + buf[pl.ds(L - k, L)]
    suf = x
    for k in (1, 2, 4, 8):
        buf[pl.ds(L, L)] = suf
        suf = suf + buf[pl.ds(L + k, L)]
    return acc, acc + suf - x


def _sc_routing_body(rs_hbm, e0_hbm, e1_hbm, w0_hbm, w1_hbm, cnt_hbm,
                     rs_v, e0_v, e1_v, w0_v, w1_v, cnt_v, sbuf):
    wid = lax.axis_index("s") * NC + lax.axis_index("c")
    base = wid * TW
    pltpu.sync_copy(rs_hbm.at[wid], rs_v)
    lane = _lane()
    zero = jnp.zeros((L,), jnp.int32)
    sbuf[pl.ds(0, L)] = zero
    sbuf[pl.ds(2 * L, L)] = zero
    cnt_acc = zero
    for c in range(TW // L):
        sl = pl.ds(c * L, L)
        v0 = rs_v[0, sl]
        i0 = jnp.zeros((L,), jnp.int32)
        for e in range(1, E):
            cand = rs_v[e, sl]
            m = cand > v0
            v0 = jnp.where(m, cand, v0)
            i0 = jnp.where(m, e, i0)
        v1 = jnp.full((L,), NEG, jnp.float32)
        i1 = jnp.zeros((L,), jnp.int32)
        for e in range(E):
            cand = rs_v[e, sl]
            m = (cand > v1) & (i0 != e)
            v1 = jnp.where(m, cand, v1)
            i1 = jnp.where(m, e, i1)
        sc = 1.0 / (v0 + v1 + 1e-08)
        e0_v[sl] = i0
        e1_v[sl] = i1
        w0_v[sl] = v0 * sc
        w1_v[sl] = v1 * sc
        for e in range(E):
            mi = jnp.where(i0 == e, 1, 0) + jnp.where(i1 == e, 1, 0)
            _, tot = _prefix_total(mi, sbuf)
            cnt_acc += jnp.where(lane == e, tot, 0)
    cnt_v[...] = cnt_acc
    pltpu.sync_copy(e0_v, e0_hbm.at[pl.ds(base, TW)])
    pltpu.sync_copy(e1_v, e1_hbm.at[pl.ds(base, TW)])
    pltpu.sync_copy(w0_v, w0_hbm.at[pl.ds(base, TW)])
    pltpu.sync_copy(w1_v, w1_hbm.at[pl.ds(base, TW)])
    pltpu.sync_copy(cnt_v, cnt_hbm.at[wid])


def _sc_dispatch_body(x_hbm, e0_hbm, e1_hbm, w0_hbm, w1_hbm, cnt_hbm, widtab_hbm,
                      xs_hbm, wp_hbm, p0_hbm, p1_hbm, cnt8_hbm,
                      cnt_m, e0_v, e1_v, p0_v, p1_v, widrow_v, sbuf,
                      w0pad, w1pad, wrow0, wrow1, xrow_v, cnt8_v, sem0, sem1):
    wid = lax.axis_index("s") * NC + lax.axis_index("c")
    base = wid * TW
    gat = pltpu.async_copy(x_hbm.at[pl.ds(base, TW)], xrow_v, sem0)
    pltpu.sync_copy(cnt_hbm, cnt_m)
    pltpu.sync_copy(e0_hbm.at[pl.ds(base, TW)], e0_v)
    pltpu.sync_copy(e1_hbm.at[pl.ds(base, TW)], e1_v)
    pltpu.sync_copy(w0_hbm.at[pl.ds(base, TW)], w0pad.at[pl.ds(0, TW)])
    pltpu.sync_copy(w1_hbm.at[pl.ds(base, TW)], w1pad.at[pl.ds(0, TW)])
    pltpu.sync_copy(widtab_hbm.at[wid], widrow_v)

    lane = _lane()
    zero = jnp.zeros((L,), jnp.int32)
    sbuf[pl.ds(0, L)] = zero
    sbuf[pl.ds(2 * L, L)] = zero
    w0pad[pl.ds(TW, L)] = jnp.zeros((L,), jnp.float32)
    w1pad[pl.ds(TW, L)] = jnp.zeros((L,), jnp.float32)

    widrow = widrow_v[...]
    total = zero
    prior = zero
    for w in range(NW):
        row = cnt_m[w, :]
        total += row
        prior += jnp.where(widrow > w, row, 0)
    cnt8_v[...] = total
    @pl.when(wid == 0)
    def _write_counts():
        pltpu.sync_copy(cnt8_v, cnt8_hbm)
    pref, _ = _prefix_total(total, sbuf)
    start = (pref - total) + prior
    start_sp = []
    for e in range(E):
        _, sp = _prefix_total(jnp.where(lane == e, start, 0), sbuf)
        start_sp.append(sp)

    carry = [zero for _ in range(E)]
    for c in range(TW // L):
        sl = pl.ds(c * L, L)
        e0c = e0_v[sl]
        e1c = e1_v[sl]
        rank0 = zero
        rank1 = zero
        s0 = zero
        s1 = zero
        for e in range(E):
            m0 = e0c == e
            m1 = e1c == e
            mi = jnp.where(m0, 1, 0) + jnp.where(m1, 1, 0)
            pref_e, tot_e = _prefix_total(mi, sbuf)
            ex = pref_e - mi + carry[e]
            rank0 += jnp.where(m0, ex, 0)
            rank1 += jnp.where(m1, ex, 0)
            s0 += jnp.where(m0, start_sp[e], 0)
            s1 += jnp.where(m1, start_sp[e], 0)
            carry[e] = carry[e] + tot_e
        p0_v[sl] = s0 + rank0
        p1_v[sl] = s1 + rank1

    for i in range(TW):
        wrow0[i, pl.ds(0, L)] = w0pad[pl.ds(i, L)]
        wrow1[i, pl.ds(0, L)] = w1pad[pl.ds(i, L)]

    pltpu.sync_copy(p0_v, p0_hbm.at[pl.ds(base, TW)])
    pltpu.sync_copy(p1_v, p1_hbm.at[pl.ds(base, TW)])
    gat.wait()
    sc0 = pltpu.async_copy(xrow_v, xs_hbm.at[p0_v], sem0)
    sc1 = pltpu.async_copy(xrow_v, xs_hbm.at[p1_v], sem1)
    sc0.wait()
    sc1.wait()
    sw0 = pltpu.async_copy(wrow0, wp_hbm.at[p0_v], sem0)
    sw1 = pltpu.async_copy(wrow1, wp_hbm.at[p1_v], sem1)
    sw0.wait()
    sw1.wait()


def _tc_gmm_body(meta_ref, xs_ref, w_ref, b_ref, wp_ref, ys_ref):
    g = pl.program_id(0)
    eg = meta_ref[0, g]
    lo = meta_ref[2, g]
    hi = meta_ref[3, g]
    x32 = xs_ref[...]
    xa = pltpu.bitcast(x32 << 16, jnp.float32)
    xb = pltpu.bitcast(x32 & jnp.int32(-65536), jnp.float32)
    y = jnp.dot(xa, w_ref[eg, : D // 2, :], preferred_element_type=jnp.float32)
    y = y + jnp.dot(xb, w_ref[eg, D // 2 :, :], preferred_element_type=jnp.float32)
    y = (y + b_ref[eg]) * wp_ref[:, 0:1]
    ri = lax.broadcasted_iota(jnp.int32, (T, 1), 0)
    keep = (ri >= lo) & (ri < hi)
    ys_ref[...] = jnp.where(keep, y, ys_ref[...])


_CH = 16
_NCH = TW // _CH


def _sc_combine_body(ys_hbm, p0_hbm, p1_hbm, out_hbm,
                     p0a, p0b, p1a, p1b, r0a, r0b, r1a, r1b, oa, ob,
                     sga, sgb, soa, sob):
    wid = lax.axis_index("s") * NC + lax.axis_index("c")
    base = wid * TW
    p0s = (p0a, p0b)
    p1s = (p1a, p1b)
    r0s = (r0a, r0b)
    r1s = (r1a, r1b)
    outs = (oa, ob)
    gsems = (sga, sgb)
    osems = (soa, sob)

    def issue(c):
        k = c % 2
        cbase = base + c * _CH
        pltpu.sync_copy(p0_hbm.at[pl.ds(cbase, _CH)], p0s[k])
        pltpu.sync_copy(p1_hbm.at[pl.ds(cbase, _CH)], p1s[k])
        g0 = pltpu.async_copy(ys_hbm.at[p0s[k]], r0s[k], gsems[k])
        g1 = pltpu.async_copy(ys_hbm.at[p1s[k]], r1s[k], gsems[k])
        return g0, g1

    pend = issue(0)
    owaits = [None, None]
    for c in range(_NCH):
        k = c % 2
        cur = pend
        if c + 1 < _NCH:
            pend = issue(c + 1)
        cur[0].wait()
        cur[1].wait()
        if owaits[k] is not None:
            owaits[k].wait()

        def tok(t, carry):
            for j in range(D // L):
                sl = pl.ds(j * L, L)
                outs[k][t, sl] = r0s[k][t, sl] + r1s[k][t, sl]
            return carry

        lax.fori_loop(0, _CH, tok, 0)
        owaits[k] = pltpu.async_copy(
            outs[k], out_hbm.at[pl.ds(base + c * _CH, _CH)], osems[k])
    for w in owaits:
        if w is not None:
            w.wait()


def _gmm_metadata(counts):
    offsets = jnp.concatenate(
        [jnp.zeros((1,), jnp.int32), jnp.cumsum(counts, dtype=jnp.int32)])
    t_start = offsets[:E] // T
    t_end = jnp.where(counts > 0, (offsets[1:] - 1) // T, t_start - 1)
    nsteps = jnp.maximum(t_end - t_start + 1, 0)
    cum = jnp.concatenate(
        [jnp.zeros((1,), jnp.int32), jnp.cumsum(nsteps, dtype=jnp.int32)])
    g = jnp.arange(G, dtype=jnp.int32)
    e_of_g = jnp.sum((g[:, None] >= cum[None, 1:]).astype(jnp.int32), axis=1)
    valid = g < cum[E]
    e_g = jnp.clip(e_of_g, 0, E - 1)
    t_g = t_start[e_g] + (g - cum[e_g])
    t_g = jnp.where(valid, t_g, NT - 1)
    lo = jnp.clip(offsets[e_g] - t_g * T, 0, T)
    hi = jnp.clip(offsets[e_g + 1] - t_g * T, 0, T)
    lo = jnp.where(valid, lo, 0)
    hi = jnp.where(valid, hi, 0)
    return jnp.stack([e_g, t_g, lo, hi]).astype(jnp.int32)


_sc_mesh = plsc.VectorSubcoreMesh(
    core_axis_name="c", subcore_axis_name="s", num_cores=NC, num_subcores=NS)

_routing = pl.kernel(
    _sc_routing_body,
    out_type=[
        jax.ShapeDtypeStruct((S,), jnp.int32),
        jax.ShapeDtypeStruct((S,), jnp.int32),
        jax.ShapeDtypeStruct((S,), jnp.float32),
        jax.ShapeDtypeStruct((S,), jnp.float32),
        jax.ShapeDtypeStruct((NW, L), jnp.int32),
    ],
    mesh=_sc_mesh,
    scratch_types=[
        pltpu.VMEM((E, TW), jnp.float32),
        pltpu.VMEM((TW,), jnp.int32),
        pltpu.VMEM((TW,), jnp.int32),
        pltpu.VMEM((TW,), jnp.float32),
        pltpu.VMEM((TW,), jnp.float32),
        pltpu.VMEM((L,), jnp.int32),
        pltpu.VMEM((3 * L,), jnp.int32),
    ],
)

_dispatch = pl.kernel(
    _sc_dispatch_body,
    out_type=[
        jax.ShapeDtypeStruct((P, D // 2), jnp.int32),
        jax.ShapeDtypeStruct((P, 128), jnp.float32),
        jax.ShapeDtypeStruct((S,), jnp.int32),
        jax.ShapeDtypeStruct((S,), jnp.int32),
        jax.ShapeDtypeStruct((L,), jnp.int32),
    ],
    mesh=_sc_mesh,
    scratch_types=[
        pltpu.VMEM((NW, L), jnp.int32),
        pltpu.VMEM((TW,), jnp.int32),
        pltpu.VMEM((TW,), jnp.int32),
        pltpu.VMEM((TW,), jnp.int32),
        pltpu.VMEM((TW,), jnp.int32),
        pltpu.VMEM((L,), jnp.int32),
        pltpu.VMEM((3 * L,), jnp.int32),
        pltpu.VMEM((TW + L,), jnp.float32),
        pltpu.VMEM((TW + L,), jnp.float32),
        pltpu.VMEM((TW, 128), jnp.float32),
        pltpu.VMEM((TW, 128), jnp.float32),
        pltpu.VMEM((TW, D // 2), jnp.int32),
        pltpu.VMEM((L,), jnp.int32),
        pltpu.SemaphoreType.DMA,
        pltpu.SemaphoreType.DMA,
    ],
)

_combine = pl.kernel(
    _sc_combine_body,
    out_type=[jax.ShapeDtypeStruct((S, D), jnp.float32)],
    mesh=_sc_mesh,
    scratch_types=[
        pltpu.VMEM((_CH,), jnp.int32),
        pltpu.VMEM((_CH,), jnp.int32),
        pltpu.VMEM((_CH,), jnp.int32),
        pltpu.VMEM((_CH,), jnp.int32),
        pltpu.VMEM((_CH, D), jnp.float32),
        pltpu.VMEM((_CH, D), jnp.float32),
        pltpu.VMEM((_CH, D), jnp.float32),
        pltpu.VMEM((_CH, D), jnp.float32),
        pltpu.VMEM((_CH, D), jnp.float32),
        pltpu.VMEM((_CH, D), jnp.float32),
        pltpu.SemaphoreType.DMA,
        pltpu.SemaphoreType.DMA,
        pltpu.SemaphoreType.DMA,
        pltpu.SemaphoreType.DMA,
    ],
)

_WIDTAB = np.tile(np.arange(NW, dtype=np.int32)[:, None], (1, L))


@jax.jit
def kernel(x, routing_scores, expert_w, expert_b):
    B = x.shape[0]
    x2 = x.reshape(S, D)
    rs_w = routing_scores.reshape(NW, TW, E).transpose(0, 2, 1)

    e0, e1, w0, w1, cnt_all = _routing(rs_w)

    x_bf = x2.astype(jnp.bfloat16)
    x_bfp = lax.bitcast_convert_type(
        jnp.stack([x_bf[:, :D // 2], x_bf[:, D // 2:]], axis=-1), jnp.int32)
    xs, wp, pos0, pos1, cnt8 = _dispatch(x_bfp, e0, e1, w0, w1, cnt_all, _WIDTAB)
    counts = cnt8[:E]
    meta = _gmm_metadata(counts)

    ys = pl.pallas_call(
        _tc_gmm_body,
        grid_spec=pltpu.PrefetchScalarGridSpec(
            num_scalar_prefetch=1,
            grid=(G,),
            in_specs=[
                pl.BlockSpec((T, D // 2), lambda g, m: (m[1, g], 0)),
                pl.BlockSpec((E, D, D), lambda g, m: (0, 0, 0)),
                pl.BlockSpec((E, 1, D), lambda g, m: (0, 0, 0)),
                pl.BlockSpec((T, 128), lambda g, m: (m[1, g], 0)),
            ],
            out_specs=pl.BlockSpec((T, D), lambda g, m: (m[1, g], 0)),
        ),
        out_shape=jax.ShapeDtypeStruct((P, D), jnp.float32),
        compiler_params=pltpu.CompilerParams(
            dimension_semantics=("arbitrary",),
        ),
    )(meta, xs, expert_w, expert_b.reshape(E, 1, D), wp)

    (out,) = _combine(ys, pos0, pos1)
    return out.reshape(B, S, D), counts


# speedup vs baseline: 1.1306x; 1.1190x over previous
import functools

import numpy as np

import jax
import jax.numpy as jnp
from jax import lax
from jax.experimental import pallas as pl
from jax.experimental.pallas import tpu as pltpu
from jax.experimental.pallas import tpu_sc as plsc

S = 2048
D = 1024
E = 8
K = 2
NC, NS, L = 2, 16, 16
NW = NC * NS
TW = S // NW

NEG = -3.0e38


def _lane():
    return lax.broadcasted_iota(jnp.int32, (L,), 0)


def _prefix_total(x, buf):
    acc = x
    for k in (1, 2, 4, 8):
        buf[pl.ds(L, L)] = acc
        acc = acc + buf[pl.ds(L - k, L)]
    suf = x
    for k in (1, 2, 4, 8):
        buf[pl.ds(L, L)] = suf
        suf = suf + buf[pl.ds(L + k, L)]
    return acc, acc + suf - x


def _sc_routing_body(rs_hbm, e0_hbm, e1_hbm, w0_hbm, w1_hbm, cnt_hbm,
                     rs_v, e0_v, e1_v, w0_v, w1_v, cnt_v, sbuf):
    wid = lax.axis_index("s") * NC + lax.axis_index("c")
    base = wid * TW
    pltpu.sync_copy(rs_hbm.at[wid], rs_v)
    lane = _lane()
    zero = jnp.zeros((L,), jnp.int32)
    sbuf[pl.ds(0, L)] = zero
    sbuf[pl.ds(2 * L, L)] = zero
    cnt_acc = zero
    for c in range(TW // L):
        sl = pl.ds(c * L, L)
        v0 = rs_v[0, sl]
        i0 = jnp.zeros((L,), jnp.int32)
        for e in range(1, E):
            cand = rs_v[e, sl]
            m = cand > v0
            v0 = jnp.where(m, cand, v0)
            i0 = jnp.where(m, e, i0)
        v1 = jnp.full((L,), NEG, jnp.float32)
        i1 = jnp.zeros((L,), jnp.int32)
        for e in range(E):
            cand = rs_v[e, sl]
            m = (cand > v1) & (i0 != e)
            v1 = jnp.where(m, cand, v1)
            i1 = jnp.where(m, e, i1)
        sc = 1.0 / (v0 + v1 + 1e-08)
        e0_v[sl] = i0
        e1_v[sl] = i1
        w0_v[sl] = v0 * sc
        w1_v[sl] = v1 * sc
        for e in range(E):
            mi = jnp.where(i0 == e, 1, 0) + jnp.where(i1 == e, 1, 0)
            _, tot = _prefix_total(mi, sbuf)
            cnt_acc += jnp.where(lane == e, tot, 0)
    cnt_v[...] = cnt_acc
    pltpu.sync_copy(e0_v, e0_hbm.at[pl.ds(base, TW)])
    pltpu.sync_copy(e1_v, e1_hbm.at[pl.ds(base, TW)])
    pltpu.sync_copy(w0_v, w0_hbm.at[pl.ds(base, TW)])
    pltpu.sync_copy(w1_v, w1_hbm.at[pl.ds(base, TW)])
    pltpu.sync_copy(cnt_v, cnt_hbm.at[wid])


_sc_mesh = plsc.VectorSubcoreMesh(
    core_axis_name="c", subcore_axis_name="s", num_cores=NC, num_subcores=NS)

_routing = pl.kernel(
    _sc_routing_body,
    out_type=[
        jax.ShapeDtypeStruct((S,), jnp.int32),
        jax.ShapeDtypeStruct((S,), jnp.int32),
        jax.ShapeDtypeStruct((S,), jnp.float32),
        jax.ShapeDtypeStruct((S,), jnp.float32),
        jax.ShapeDtypeStruct((NW, L), jnp.int32),
    ],
    mesh=_sc_mesh,
    scratch_types=[
        pltpu.VMEM((E, TW), jnp.float32),
        pltpu.VMEM((TW,), jnp.int32),
        pltpu.VMEM((TW,), jnp.int32),
        pltpu.VMEM((TW,), jnp.float32),
        pltpu.VMEM((TW,), jnp.float32),
        pltpu.VMEM((L,), jnp.int32),
        pltpu.VMEM((3 * L,), jnp.int32),
    ],
)


def _tc_moe_body(e0_ref, e1_ref, w0_ref, w1_ref, x_ref, w_ref, b_ref, out_ref):
    e = pl.program_id(0)
    x = x_ref[...]
    y = jnp.dot(x, w_ref[0], preferred_element_type=jnp.float32)
    col = jnp.where(e0_ref[...] == e, w0_ref[...], 0.0)
    col = col + jnp.where(e1_ref[...] == e, w1_ref[...], 0.0)
    contrib = col * (y + b_ref[0])

    @pl.when(e == 0)
    def _init():
        out_ref[...] = contrib

    @pl.when(e != 0)
    def _acc():
        out_ref[...] += contrib


@jax.jit
def kernel(x, routing_scores, expert_w, expert_b):
    B = x.shape[0]
    x2 = x.reshape(S, D)
    rs_w = routing_scores.reshape(NW, TW, E).transpose(0, 2, 1)

    e0, e1, w0, w1, cnt_all = _routing(rs_w)
    counts = jnp.sum(cnt_all, axis=0)[:E].astype(jnp.int32)

    out = pl.pallas_call(
        _tc_moe_body,
        grid=(E,),
        in_specs=[
            pl.BlockSpec((S, 1), lambda e: (0, 0)),
            pl.BlockSpec((S, 1), lambda e: (0, 0)),
            pl.BlockSpec((S, 1), lambda e: (0, 0)),
            pl.BlockSpec((S, 1), lambda e: (0, 0)),
            pl.BlockSpec((S, D), lambda e: (0, 0)),
            pl.BlockSpec((1, D, D), lambda e: (e, 0, 0)),
            pl.BlockSpec((1, 1, D), lambda e: (e, 0, 0)),
        ],
        out_specs=pl.BlockSpec((S, D), lambda e: (0, 0)),
        out_shape=jax.ShapeDtypeStruct((S, D), jnp.float32),
        compiler_params=pltpu.CompilerParams(
            dimension_semantics=("arbitrary",),
        ),
    )(e0.reshape(S, 1), e1.reshape(S, 1), w0.reshape(S, 1), w1.reshape(S, 1),
      x2, expert_w, expert_b.reshape(E, 1, D))

    return out.reshape(B, S, D), counts


# speedup vs baseline: 1.2280x; 1.0861x over previous
import functools

import numpy as np

import jax
import jax.numpy as jnp
from jax import lax
from jax.experimental import pallas as pl
from jax.experimental.pallas import tpu as pltpu
from jax.experimental.pallas import tpu_sc as plsc

S = 2048
D = 1024
E = 8
K = 2
NC, NS, L = 2, 16, 16
NW = NC * NS
TW = S // NW

NEG = -3.0e38


def _lane():
    return lax.broadcasted_iota(jnp.int32, (L,), 0)


def _prefix_total(x, buf):
    acc = x
    for k in (1, 2, 4, 8):
        buf[pl.ds(L, L)] = acc
        acc = acc + buf[pl.ds(L - k, L)]
    suf = x
    for k in (1, 2, 4, 8):
        buf[pl.ds(L, L)] = suf
        suf = suf + buf[pl.ds(L + k, L)]
    return acc, acc + suf - x


def _sc_routing_body(rs_hbm, ms_hbm, cnt_hbm,
                     rs_v, ms_v, cnt_v, sbuf):
    wid = lax.axis_index("s") * NC + lax.axis_index("c")
    base = wid * TW
    pltpu.sync_copy(rs_hbm.at[wid], rs_v)
    lane = _lane()
    zero = jnp.zeros((L,), jnp.int32)
    sbuf[pl.ds(0, L)] = zero
    sbuf[pl.ds(2 * L, L)] = zero
    cnt_acc = zero
    for c in range(TW // L):
        sl = pl.ds(c * L, L)
        v0 = rs_v[0, sl]
        i0 = jnp.zeros((L,), jnp.int32)
        for e in range(1, E):
            cand = rs_v[e, sl]
            m = cand > v0
            v0 = jnp.where(m, cand, v0)
            i0 = jnp.where(m, e, i0)
        v1 = jnp.full((L,), NEG, jnp.float32)
        i1 = jnp.zeros((L,), jnp.int32)
        for e in range(E):
            cand = rs_v[e, sl]
            m = (cand > v1) & (i0 != e)
            v1 = jnp.where(m, cand, v1)
            i1 = jnp.where(m, e, i1)
        sc = 1.0 / (v0 + v1 + 1e-08)
        wa = v0 * sc
        wb = v1 * sc
        for e in range(E):
            mi = jnp.where(i0 == e, 1, 0) + jnp.where(i1 == e, 1, 0)
            _, tot = _prefix_total(mi, sbuf)
            cnt_acc += jnp.where(lane == e, tot, 0)
            ms_v[e, sl] = (jnp.where(i0 == e, wa, 0.0)
                           + jnp.where(i1 == e, wb, 0.0))
    cnt_v[...] = cnt_acc
    for e in range(E):
        pltpu.sync_copy(ms_v.at[e], ms_hbm.at[e, pl.ds(base, TW)])
    pltpu.sync_copy(cnt_v, cnt_hbm.at[wid])


_sc_mesh = plsc.VectorSubcoreMesh(
    core_axis_name="c", subcore_axis_name="s", num_cores=NC, num_subcores=NS)

_routing = pl.kernel(
    _sc_routing_body,
    out_type=[
        jax.ShapeDtypeStruct((E, S), jnp.float32),
        jax.ShapeDtypeStruct((NW, L), jnp.int32),
    ],
    mesh=_sc_mesh,
    scratch_types=[
        pltpu.VMEM((E, TW), jnp.float32),
        pltpu.VMEM((E, TW), jnp.float32),
        pltpu.VMEM((L,), jnp.int32),
        pltpu.VMEM((3 * L,), jnp.int32),
    ],
)


def _tc_moe_body(ms_ref, x_ref, w_ref, b_ref, out_ref):
    e = pl.program_id(0)
    x = x_ref[...]
    y = jnp.dot(x, w_ref[0], preferred_element_type=jnp.float32)
    ms = ms_ref[...]
    lane = lax.broadcasted_iota(jnp.int32, ms.shape, 1)
    col = jnp.sum(jnp.where(lane == e, ms, 0.0), axis=1, keepdims=True)
    contrib = col * (y + b_ref[0])

    @pl.when(e == 0)
    def _init():
        out_ref[...] = contrib

    @pl.when(e != 0)
    def _acc():
        out_ref[...] += contrib


@jax.jit
def kernel(x, routing_scores, expert_w, expert_b):
    B = x.shape[0]
    x2 = x.reshape(S, D)
    rs_w = routing_scores.reshape(NW, TW, E).transpose(0, 2, 1)

    ms_t, cnt_all = _routing(rs_w)
    counts = jnp.sum(cnt_all, axis=0)[:E].astype(jnp.int32)
    ms = ms_t.T

    out = pl.pallas_call(
        _tc_moe_body,
        grid=(E,),
        in_specs=[
            pl.BlockSpec((S, E), lambda e: (0, 0)),
            pl.BlockSpec((S, D), lambda e: (0, 0)),
            pl.BlockSpec((1, D, D), lambda e: (e, 0, 0)),
            pl.BlockSpec((1, 1, D), lambda e: (e, 0, 0)),
        ],
        out_specs=pl.BlockSpec((S, D), lambda e: (0, 0)),
        out_shape=jax.ShapeDtypeStruct((S, D), jnp.float32),
        compiler_params=pltpu.CompilerParams(
            dimension_semantics=("arbitrary",),
        ),
    )(ms, x2, expert_w, expert_b.reshape(E, 1, D))

    return out.reshape(B, S, D), counts
